# baseline (device time: 87669 ns/iter reference)
import jax
import jax.numpy as jnp
from jax import lax
from jax.experimental import pallas as pl
from jax.experimental.pallas import tpu as pltpu

N_DEV = 4
B, SQ, DM = 2, 256, 512
HQ, DH = 16, 64
HP = HQ // N_DEV
SKV = 256
NCHUNK = 2
WIN = 128
F32 = jnp.float32


def kernel(x, Wq, K_ext, V_ext, Wo):
    def body(
        x_ref, wq_ref, k_ref, v_ref, wo_ref, out_ref,
        kbuf, vbuf, ksend, vsend, abuf,
        ksend_sems, vsend_sems, krecv_sems, vrecv_sems,
        asend_sems, arecv_sems,
    ):
        my = lax.axis_index("i")

        bsem = pltpu.get_barrier_semaphore()
        for off in range(1, N_DEV):
            pl.semaphore_signal(
                bsem, inc=1,
                device_id=((my + off) % N_DEV,),
                device_id_type=pl.DeviceIdType.MESH,
            )
        pl.semaphore_wait(bsem, N_DEV - 1)

        for sender in range(NCHUNK):

            @pl.when(my == sender)
            def _(sender=sender):
                kv_k = k_ref[...]
                kv_v = v_ref[...]
                kbuf[sender] = kv_k[:, :, sender * HP:(sender + 1) * HP, :]
                vbuf[sender] = kv_v[:, :, sender * HP:(sender + 1) * HP, :]
                rdmas = []
                for dst in range(N_DEV):
                    if dst == sender:
                        continue
                    ksend[dst] = kv_k[:, :, dst * HP:(dst + 1) * HP, :]
                    vsend[dst] = kv_v[:, :, dst * HP:(dst + 1) * HP, :]
                    for src_r, dst_r, ss, rs in (
                        (ksend, kbuf, ksend_sems, krecv_sems),
                        (vsend, vbuf, vsend_sems, vrecv_sems),
                    ):
                        r = pltpu.make_async_remote_copy(
                            src_ref=src_r.at[dst],
                            dst_ref=dst_r.at[sender],
                            send_sem=ss.at[dst],
                            recv_sem=rs.at[sender],
                            device_id=(dst,),
                            device_id_type=pl.DeviceIdType.MESH,
                        )
                        r.start()
                        rdmas.append(r)
                other = 1 - sender
                for buf, rs in ((kbuf, krecv_sems), (vbuf, vrecv_sems)):
                    pltpu.make_async_remote_copy(
                        src_ref=buf.at[other], dst_ref=buf.at[other],
                        send_sem=ksend_sems.at[sender],
                        recv_sem=rs.at[other],
                        device_id=(other,),
                        device_id_type=pl.DeviceIdType.MESH,
                    ).wait_recv()
                for r in rdmas:
                    r.wait_send()

        @pl.when(my >= NCHUNK)
        def _():
            for c in range(NCHUNK):
                for buf, rs in ((kbuf, krecv_sems), (vbuf, vrecv_sems)):
                    pltpu.make_async_remote_copy(
                        src_ref=buf.at[c], dst_ref=buf.at[c],
                        send_sem=ksend_sems.at[c],
                        recv_sem=rs.at[c],
                        device_id=(c,),
                        device_id_type=pl.DeviceIdType.MESH,
                    ).wait_recv()

        qflat = lax.dot_general(
            x_ref[...].reshape(B * SQ, DM), wq_ref[...],
            (((1,), (0,)), ((), ())), preferred_element_type=F32,
        )
        kcat = jnp.concatenate([kbuf[0], kbuf[1]], axis=1)
        vcat = jnp.concatenate([vbuf[0], vbuf[1]], axis=1)

        qi = lax.broadcasted_iota(jnp.int32, (SQ, NCHUNK * SKV), 0)
        ki = lax.broadcasted_iota(jnp.int32, (SQ, NCHUNK * SKV), 1)
        mask = jnp.abs(qi - ki) <= WIN

        rows = []
        for b in range(B):
            heads = []
            for h in range(HP):
                q_bh = qflat[b * SQ:(b + 1) * SQ, h * DH:(h + 1) * DH]
                k_bh = kcat[b, :, h, :]
                v_bh = vcat[b, :, h, :]
                s = lax.dot_general(
                    q_bh, k_bh, (((1,), (1,)), ((), ())),
                    preferred_element_type=F32,
                ) * 0.125
                s = jnp.where(mask, s, F32(-1e9))
                s = s - jnp.max(s, axis=-1, keepdims=True)
                w = jnp.exp(s)
                w = w / jnp.sum(w, axis=-1, keepdims=True)
                heads.append(lax.dot_general(
                    w, v_bh, (((1,), (0,)), ((), ())),
                    preferred_element_type=F32,
                ))
            rows.append(jnp.concatenate(heads, axis=1))
        ctxflat = jnp.concatenate(rows, axis=0)
        partial = lax.dot_general(
            ctxflat, wo_ref[...], (((1,), (0,)), ((), ())),
            preferred_element_type=F32,
        )

        for me in range(N_DEV):

            @pl.when(my == me)
            def _(me=me):
                abuf[me] = partial
                rdmas = []
                for dst in range(N_DEV):
                    if dst == me:
                        continue
                    r = pltpu.make_async_remote_copy(
                        src_ref=abuf.at[me], dst_ref=abuf.at[me],
                        send_sem=asend_sems.at[dst],
                        recv_sem=arecv_sems.at[me],
                        device_id=(dst,),
                        device_id_type=pl.DeviceIdType.MESH,
                    )
                    r.start()
                    rdmas.append(r)
                for src in range(N_DEV):
                    if src == me:
                        continue
                    pltpu.make_async_remote_copy(
                        src_ref=abuf.at[src], dst_ref=abuf.at[src],
                        send_sem=asend_sems.at[src],
                        recv_sem=arecv_sems.at[src],
                        device_id=(src,),
                        device_id_type=pl.DeviceIdType.MESH,
                    ).wait_recv()
                for r in rdmas:
                    r.wait_send()

        total = abuf[0] + abuf[1] + abuf[2] + abuf[3]
        out_ref[...] = total.reshape(B, SQ, DM)

    return pl.pallas_call(
        body,
        out_shape=jax.ShapeDtypeStruct((B, SQ, DM), F32),
        in_specs=[pl.BlockSpec(memory_space=pltpu.VMEM)] * 5,
        out_specs=pl.BlockSpec(memory_space=pltpu.VMEM),
        scratch_shapes=[
            pltpu.VMEM((NCHUNK, B, SKV, HP, DH), F32),
            pltpu.VMEM((NCHUNK, B, SKV, HP, DH), F32),
            pltpu.VMEM((N_DEV, B, SKV, HP, DH), F32),
            pltpu.VMEM((N_DEV, B, SKV, HP, DH), F32),
            pltpu.VMEM((N_DEV, B * SQ, DM), F32),
            pltpu.SemaphoreType.DMA((N_DEV,)),
            pltpu.SemaphoreType.DMA((N_DEV,)),
            pltpu.SemaphoreType.DMA((NCHUNK,)),
            pltpu.SemaphoreType.DMA((NCHUNK,)),
            pltpu.SemaphoreType.DMA((N_DEV,)),
            pltpu.SemaphoreType.DMA((N_DEV,)),
        ],
        compiler_params=pltpu.CompilerParams(collective_id=0),
    )(x, Wq, K_ext, V_ext, Wo)


# device time: 78047 ns/iter; 1.1233x vs baseline; 1.1233x over previous
import jax
import jax.numpy as jnp
from jax import lax
from jax.experimental import pallas as pl
from jax.experimental.pallas import tpu as pltpu

N_DEV = 4
B, SQ, DM = 2, 256, 512
HQ, DH = 16, 64
HP = HQ // N_DEV
SKV = 256
C1 = 128
KV = SKV + C1
RB = (B * SQ) // N_DEV
WIN = 128
F32 = jnp.float32

_CHUNKS = ((0, SKV), (1, C1))


def kernel(x, Wq, K_ext, V_ext, Wo):
    def body(
        x_ref, wq_ref, k_ref, v_ref, wo_ref, out_ref,
        kbuf0, kbuf1, vbuf0, vbuf1, psend, rsbuf, obuf,
        lsems, ksend_sems, vsend_sems, krecv_sems, vrecv_sems,
        rs_send_sems, rs_recv_sems, ag_send_sems, ag_recv_sems,
    ):
        my = lax.axis_index("i")

        def kvbufs(sender):
            return (kbuf0, vbuf0) if sender == 0 else (kbuf1, vbuf1)

        def phase1_rdmas(sender, rows):
            kb, vb = kvbufs(sender)
            out = []
            for dst in range(N_DEV):
                if dst == sender:
                    continue
                for src_r, dst_r, ss, rs in (
                    (k_ref, kb, ksend_sems, krecv_sems),
                    (v_ref, vb, vsend_sems, vrecv_sems),
                ):
                    out.append(pltpu.make_async_remote_copy(
                        src_ref=src_r.at[:, pl.ds(0, rows),
                                         pl.ds(dst * HP, HP), :],
                        dst_ref=dst_r,
                        send_sem=ss.at[dst],
                        recv_sem=rs.at[sender],
                        device_id=(dst,),
                        device_id_type=pl.DeviceIdType.MESH,
                    ))
            return out

        def local_copies(sender, rows):
            kb, vb = kvbufs(sender)
            return [
                pltpu.make_async_copy(
                    r.at[:, pl.ds(0, rows), pl.ds(sender * HP, HP), :],
                    b, lsems.at[i],
                )
                for i, (r, b) in enumerate(((k_ref, kb), (v_ref, vb)))
            ]

        def recv_wait(buf, sems, slot):
            pltpu.make_async_remote_copy(
                src_ref=buf, dst_ref=buf,
                send_sem=ksend_sems.at[0], recv_sem=sems.at[slot],
                device_id=(0,), device_id_type=pl.DeviceIdType.MESH,
            ).wait_recv()

        bsem = pltpu.get_barrier_semaphore()
        for off in range(1, N_DEV):
            pl.semaphore_signal(
                bsem, inc=1,
                device_id=((my + off) % N_DEV,),
                device_id_type=pl.DeviceIdType.MESH,
            )
        pl.semaphore_wait(bsem, N_DEV - 1)

        for sender, rows in _CHUNKS:

            @pl.when(my == sender)
            def _(sender=sender, rows=rows):
                for c in local_copies(sender, rows):
                    c.start()
                for r in phase1_rdmas(sender, rows):
                    r.start()

        qflat = lax.dot_general(
            x_ref[...].reshape(B * SQ, DM), wq_ref[...],
            (((1,), (0,)), ((), ())), preferred_element_type=F32,
        )
        qi = lax.broadcasted_iota(jnp.int32, (SQ, KV), 0)
        ki = lax.broadcasted_iota(jnp.int32, (SQ, KV), 1)
        mask = jnp.abs(qi - ki) <= WIN

        for sender, rows in _CHUNKS:

            @pl.when(my == sender)
            def _(sender=sender, rows=rows):
                for c in local_copies(sender, rows):
                    c.wait()
                other = 1 - sender
                okb, ovb = kvbufs(other)
                recv_wait(okb, krecv_sems, other)
                recv_wait(ovb, vrecv_sems, other)
                for r in phase1_rdmas(sender, rows):
                    r.wait_send()

        @pl.when(my >= 2)
        def _():
            for c in range(2):
                kb, vb = kvbufs(c)
                recv_wait(kb, krecv_sems, c)
                recv_wait(vb, vrecv_sems, c)

        kcat = jnp.concatenate([kbuf0[...], kbuf1[...]], axis=1)
        vcat = jnp.concatenate([vbuf0[...], vbuf1[...]], axis=1)

        rows_out = []
        for b in range(B):
            heads = []
            for h in range(HP):
                q_bh = qflat[b * SQ:(b + 1) * SQ, h * DH:(h + 1) * DH]
                k_bh = kcat[b, :, h, :]
                v_bh = vcat[b, :, h, :]
                s = lax.dot_general(
                    q_bh, k_bh, (((1,), (1,)), ((), ())),
                    preferred_element_type=F32,
                ) * 0.125
                s = jnp.where(mask, s, F32(-1e9))
                s = s - jnp.max(s, axis=-1, keepdims=True)
                w = jnp.exp(s)
                w = w / jnp.sum(w, axis=-1, keepdims=True)
                heads.append(lax.dot_general(
                    w, v_bh, (((1,), (0,)), ((), ())),
                    preferred_element_type=F32,
                ))
            rows_out.append(jnp.concatenate(heads, axis=1))
        ctxflat = jnp.concatenate(rows_out, axis=0)
        partial = lax.dot_general(
            ctxflat, wo_ref[...], (((1,), (0,)), ((), ())),
            preferred_element_type=F32,
        )

        psend[...] = partial.reshape(N_DEV, RB, DM)
        for me in range(N_DEV):

            @pl.when(my == me)
            def _(me=me):
                others = [d for d in range(N_DEV) if d != me]
                rsbuf[me] = psend[me]

                def rs_rdma(dst):
                    return pltpu.make_async_remote_copy(
                        src_ref=psend.at[dst], dst_ref=rsbuf.at[me],
                        send_sem=rs_send_sems.at[dst],
                        recv_sem=rs_recv_sems.at[me],
                        device_id=(dst,),
                        device_id_type=pl.DeviceIdType.MESH,
                    )

                def ag_rdma(dst):
                    return pltpu.make_async_remote_copy(
                        src_ref=obuf.at[me], dst_ref=obuf.at[me],
                        send_sem=ag_send_sems.at[dst],
                        recv_sem=ag_recv_sems.at[me],
                        device_id=(dst,),
                        device_id_type=pl.DeviceIdType.MESH,
                    )

                for dst in others:
                    rs_rdma(dst).start()
                for src in others:
                    recv_wait(rsbuf.at[src], rs_recv_sems, src)
                obuf[me] = rsbuf[0] + rsbuf[1] + rsbuf[2] + rsbuf[3]
                for dst in others:
                    ag_rdma(dst).start()
                for src in others:
                    recv_wait(obuf.at[src], ag_recv_sems, src)
                for dst in others:
                    rs_rdma(dst).wait_send()
                    ag_rdma(dst).wait_send()

        for j in range(N_DEV):
            out_ref[j // 2, pl.ds((j % 2) * RB, RB), :] = obuf[j]

    return pl.pallas_call(
        body,
        out_shape=jax.ShapeDtypeStruct((B, SQ, DM), F32),
        in_specs=[
            pl.BlockSpec(memory_space=pltpu.VMEM),
            pl.BlockSpec(memory_space=pltpu.VMEM),
            pl.BlockSpec(memory_space=pltpu.MemorySpace.HBM),
            pl.BlockSpec(memory_space=pltpu.MemorySpace.HBM),
            pl.BlockSpec(memory_space=pltpu.VMEM),
        ],
        out_specs=pl.BlockSpec(memory_space=pltpu.VMEM),
        scratch_shapes=[
            pltpu.VMEM((B, SKV, HP, DH), F32),
            pltpu.VMEM((B, C1, HP, DH), F32),
            pltpu.VMEM((B, SKV, HP, DH), F32),
            pltpu.VMEM((B, C1, HP, DH), F32),
            pltpu.VMEM((N_DEV, RB, DM), F32),
            pltpu.VMEM((N_DEV, RB, DM), F32),
            pltpu.VMEM((N_DEV, RB, DM), F32),
            pltpu.SemaphoreType.DMA((2,)),
            pltpu.SemaphoreType.DMA((N_DEV,)),
            pltpu.SemaphoreType.DMA((N_DEV,)),
            pltpu.SemaphoreType.DMA((2,)),
            pltpu.SemaphoreType.DMA((2,)),
            pltpu.SemaphoreType.DMA((N_DEV,)),
            pltpu.SemaphoreType.DMA((N_DEV,)),
            pltpu.SemaphoreType.DMA((N_DEV,)),
            pltpu.SemaphoreType.DMA((N_DEV,)),
        ],
        compiler_params=pltpu.CompilerParams(collective_id=0),
    )(x, Wq, K_ext, V_ext, Wo)


# device time: 44933 ns/iter; 1.9511x vs baseline; 1.7370x over previous
import os

import jax
import jax.numpy as jnp
from jax import lax
from jax.experimental import pallas as pl
from jax.experimental.pallas import tpu as pltpu

_ABLATE = set(os.environ.get("ABLATE", "").split(","))

N_DEV = 4
B, SQ, DM = 2, 256, 512
HQ, DH = 16, 64
HP = HQ // N_DEV
SKV = 256
C1 = 128
KV = SKV + C1
RB = (B * SQ) // N_DEV
WIN = 128
F32 = jnp.float32
BF16 = jnp.bfloat16
FP8 = jnp.bfloat16

_CHUNKS = ((0, SKV), (1, C1))


def kernel(x, Wq, K_ext, V_ext, Wo):
    def body(
        x_ref, wq_ref, k_ref, v_ref, wo_ref, out_ref,
        kvstage, ksend, vsend, kbuf0, kbuf1, vbuf0, vbuf1,
        rk2, rv2, rk3, rv3,
        psend, rsbuf, obuf,
        stage_sems, ksend_sems, vsend_sems, krecv_sems, vrecv_sems,
        relay_sems, fwd_send_sems,
        rs_send_sems, rs_recv_sems, ag_send_sems, ag_recv_sems,
    ):
        my = lax.axis_index("i")

        def chunk_bufs(c):
            return (kbuf0, vbuf0) if c == 0 else (kbuf1, vbuf1)

        def p1_rdmas(sender, rows, dst):
            kb, vb = chunk_bufs(sender)
            out = []
            for send_buf, dst_buf, ss, rs in (
                (ksend, kb, ksend_sems, krecv_sems),
                (vsend, vb, vsend_sems, vrecv_sems),
            ):
                out.append(pltpu.make_async_remote_copy(
                    src_ref=send_buf.at[dst, :, pl.ds(0, rows), :, :],
                    dst_ref=dst_buf,
                    send_sem=ss.at[dst],
                    recv_sem=rs.at[sender],
                    device_id=(dst,),
                    device_id_type=pl.DeviceIdType.MESH,
                ))
            return out

        def stage_copies():
            return [
                pltpu.make_async_copy(r, kvstage.at[i], stage_sems.at[i])
                for i, r in enumerate((k_ref, v_ref))
            ]

        def _rc(src, dst, ss, rs, dev):
            return pltpu.make_async_remote_copy(
                src_ref=src, dst_ref=dst, send_sem=ss, recv_sem=rs,
                device_id=(dev,), device_id_type=pl.DeviceIdType.MESH,
            )

        def relay_rdmas(sender):
            if sender == 0:
                return [
                    _rc(ksend.at[2], rk2, ksend_sems.at[2],
                        relay_sems.at[0], 1),
                    _rc(vsend.at[2], rv2, vsend_sems.at[2],
                        relay_sems.at[1], 3),
                ]
            return [
                _rc(ksend.at[3, :, pl.ds(0, C1), :, :], rk3,
                    ksend_sems.at[3], relay_sems.at[2], 2),
                _rc(vsend.at[3, :, pl.ds(0, C1), :, :], rv3,
                    vsend_sems.at[3], relay_sems.at[3], 2),
            ]

        _FWD = {
            1: [(rk2, kbuf0, 0, krecv_sems, 0, 2)],
            3: [(rv2, vbuf0, 0, vrecv_sems, 0, 2)],
            2: [(rk3, kbuf1, 0, krecv_sems, 1, 3),
                (rv3, vbuf1, 1, vrecv_sems, 1, 3)],
        }
        _FWD_WAIT = {1: [(rk2, 0)], 3: [(rv2, 1)], 2: [(rk3, 2), (rv3, 3)]}

        def fwd_rdmas(me):
            return [
                _rc(src, dst, fwd_send_sems.at[si], rs.at[slot], dev)
                for src, dst, si, rs, slot, dev in _FWD[me]
            ]

        def recv_wait(buf, sems, slot):
            pltpu.make_async_remote_copy(
                src_ref=buf, dst_ref=buf,
                send_sem=ksend_sems.at[0], recv_sem=sems.at[slot],
                device_id=(0,), device_id_type=pl.DeviceIdType.MESH,
            ).wait_recv()

        if "p1" not in _ABLATE:

            @pl.when(my < 2)
            def _():
                for c in stage_copies():
                    c.start()

        bsem = pltpu.get_barrier_semaphore()
        for off in range(1, N_DEV):
            pl.semaphore_signal(
                bsem, inc=1,
                device_id=((my + off) % N_DEV,),
                device_id_type=pl.DeviceIdType.MESH,
            )
        pl.semaphore_wait(bsem, N_DEV - 1)

        for sender, rows in _CHUNKS if "p1" not in _ABLATE else ():

            @pl.when(my == sender)
            def _(sender=sender, rows=rows):
                for c in stage_copies():
                    c.wait()
                kfull = kvstage[0, :, :rows].astype(BF16)
                vfull = kvstage[1, :, :rows].astype(FP8)
                mk, mv = chunk_bufs(sender)
                mk[...] = kfull[:, :, sender * HP:(sender + 1) * HP, :]
                mv[...] = vfull[:, :, sender * HP:(sender + 1) * HP, :]
                diag = (sender + 2) % N_DEV
                hs = slice(diag * HP, (diag + 1) * HP)
                ksend[diag, :, pl.ds(0, rows), :, :] = kfull[:, :, hs, :]
                vsend[diag, :, pl.ds(0, rows), :, :] = vfull[:, :, hs, :]
                if "p1send" not in _ABLATE:
                    for r in relay_rdmas(sender):
                        r.start()
                for dst in range(N_DEV):
                    if dst == sender or dst == diag:
                        continue
                    hs = slice(dst * HP, (dst + 1) * HP)
                    ksend[dst, :, pl.ds(0, rows), :, :] = kfull[:, :, hs, :]
                    vsend[dst, :, pl.ds(0, rows), :, :] = vfull[:, :, hs, :]
                    if "p1send" not in _ABLATE:
                        for r in p1_rdmas(sender, rows, dst):
                            r.start()

        if "p1" not in _ABLATE and "p1send" not in _ABLATE:
            for fw in (1, 3, 2):

                @pl.when(my == fw)
                def _(fw=fw):
                    for buf, slot in _FWD_WAIT[fw]:
                        pltpu.make_async_remote_copy(
                            src_ref=buf, dst_ref=buf,
                            send_sem=ksend_sems.at[0],
                            recv_sem=relay_sems.at[slot],
                            device_id=(0,),
                            device_id_type=pl.DeviceIdType.MESH,
                        ).wait_recv()
                    for r in fwd_rdmas(fw):
                        r.start()

        qflat = lax.dot_general(
            x_ref[...].reshape(B * SQ, DM), wq_ref[...],
            (((1,), (0,)), ((), ())), preferred_element_type=F32,
        )
        qi = lax.broadcasted_iota(jnp.int32, (SQ, KV), 0)
        ki = lax.broadcasted_iota(jnp.int32, (SQ, KV), 1)
        mask = jnp.abs(qi - ki) <= WIN

        live_p1 = "p1" not in _ABLATE and "p1send" not in _ABLATE
        if live_p1:

            @pl.when(my != 0)
            def _():
                recv_wait(kbuf0, krecv_sems, 0)
                recv_wait(vbuf0, vrecv_sems, 0)

        qbf = qflat.astype(BF16)
        k0 = kbuf0[...]
        bh = [(b, h) for b in range(B) for h in range(HP)]
        q_bhs = {
            (b, h): qbf[b * SQ:(b + 1) * SQ, h * DH:(h + 1) * DH]
            for b, h in bh
        }
        s0 = {
            (b, h): lax.dot_general(
                q_bhs[b, h], k0[b, :, h, :], (((1,), (1,)), ((), ())),
                preferred_element_type=F32,
            )
            for b, h in bh
        }

        if live_p1:

            @pl.when(my != 1)
            def _():
                recv_wait(kbuf1, krecv_sems, 1)
                recv_wait(vbuf1, vrecv_sems, 1)

            for sender, rows in _CHUNKS:

                @pl.when(my == sender)
                def _(sender=sender, rows=rows):
                    diag = (sender + 2) % N_DEV
                    for dst in range(N_DEV):
                        if dst != sender and dst != diag:
                            for r in p1_rdmas(sender, rows, dst):
                                r.wait_send()
                    for r in relay_rdmas(sender):
                        r.wait_send()

            for fw in (1, 3, 2):

                @pl.when(my == fw)
                def _(fw=fw):
                    for r in fwd_rdmas(fw):
                        r.wait_send()

        k1 = kbuf1[...]
        v0 = vbuf0[...].astype(BF16)
        v1 = vbuf1[...].astype(BF16)
        wo = wo_ref[...]

        def rs_rdma(me, j):
            return pltpu.make_async_remote_copy(
                src_ref=psend.at[j], dst_ref=rsbuf.at[me],
                send_sem=rs_send_sems.at[j],
                recv_sem=rs_recv_sems.at[me],
                device_id=(j,),
                device_id_type=pl.DeviceIdType.MESH,
            )

        rows_out = []
        for b in range(B):
            heads = []
            for h in range(HP):
                s1 = lax.dot_general(
                    q_bhs[b, h], k1[b, :, h, :], (((1,), (1,)), ((), ())),
                    preferred_element_type=F32,
                )
                s = jnp.concatenate([s0[b, h], s1], axis=1) * 0.125
                s = jnp.where(mask, s, F32(-1e9))
                s = s - jnp.max(s, axis=-1, keepdims=True)
                w = jnp.exp(s)
                w = (w / jnp.sum(w, axis=-1, keepdims=True)).astype(BF16)
                ctx = lax.dot_general(
                    w[:, :SKV], v0[b, :, h, :], (((1,), (0,)), ((), ())),
                    preferred_element_type=F32,
                ) + lax.dot_general(
                    w[:, SKV:], v1[b, :, h, :], (((1,), (0,)), ((), ())),
                    preferred_element_type=F32,
                )
                heads.append(ctx)
            rows_out.append(jnp.concatenate(heads, axis=1))
        partial = lax.dot_general(
            jnp.concatenate(rows_out, axis=0), wo,
            (((1,), (0,)), ((), ())), preferred_element_type=F32,
        )
        psend[...] = partial.reshape(N_DEV, RB, DM).astype(BF16)

        for me in range(N_DEV):

            @pl.when(my == me)
            def _(me=me):
                others = [(me + 2) % N_DEV] + [
                    d for d in range(N_DEV)
                    if d != me and d != (me + 2) % N_DEV
                ]
                rsbuf[me] = psend[me]
                if "p3" not in _ABLATE:
                    for j in others:
                        rs_rdma(me, j).start()

                def ag_rdma(dst):
                    return pltpu.make_async_remote_copy(
                        src_ref=obuf.at[me], dst_ref=obuf.at[me],
                        send_sem=ag_send_sems.at[dst],
                        recv_sem=ag_recv_sems.at[me],
                        device_id=(dst,),
                        device_id_type=pl.DeviceIdType.MESH,
                    )

                if "p3" not in _ABLATE:
                    for src in others:
                        recv_wait(rsbuf.at[src], rs_recv_sems, src)
                obuf[me] = (
                    rsbuf[0].astype(F32) + rsbuf[1].astype(F32)
                    + rsbuf[2].astype(F32) + rsbuf[3].astype(F32)
                ).astype(BF16)
                if "p3" not in _ABLATE:
                    for dst in others:
                        ag_rdma(dst).start()
                    for src in others:
                        recv_wait(obuf.at[src], ag_recv_sems, src)
                    for j in range(N_DEV):
                        if j != me:
                            rs_rdma(me, j).wait_send()
                    for dst in others:
                        ag_rdma(dst).wait_send()

        for j in range(N_DEV):
            out_ref[j // 2, pl.ds((j % 2) * RB, RB), :] = obuf[j].astype(F32)

    return pl.pallas_call(
        body,
        out_shape=jax.ShapeDtypeStruct((B, SQ, DM), F32),
        in_specs=[
            pl.BlockSpec(memory_space=pltpu.VMEM),
            pl.BlockSpec(memory_space=pltpu.VMEM),
            pl.BlockSpec(memory_space=pltpu.MemorySpace.HBM),
            pl.BlockSpec(memory_space=pltpu.MemorySpace.HBM),
            pl.BlockSpec(memory_space=pltpu.VMEM),
        ],
        out_specs=pl.BlockSpec(memory_space=pltpu.VMEM),
        scratch_shapes=[
            pltpu.VMEM((2, B, SKV, HQ, DH), F32),
            pltpu.VMEM((N_DEV, B, SKV, HP, DH), BF16),
            pltpu.VMEM((N_DEV, B, SKV, HP, DH), FP8),
            pltpu.VMEM((B, SKV, HP, DH), BF16),
            pltpu.VMEM((B, C1, HP, DH), BF16),
            pltpu.VMEM((B, SKV, HP, DH), FP8),
            pltpu.VMEM((B, C1, HP, DH), FP8),
            pltpu.VMEM((B, SKV, HP, DH), BF16),
            pltpu.VMEM((B, SKV, HP, DH), FP8),
            pltpu.VMEM((B, C1, HP, DH), BF16),
            pltpu.VMEM((B, C1, HP, DH), FP8),
            pltpu.VMEM((N_DEV, RB, DM), BF16),
            pltpu.VMEM((N_DEV, RB, DM), BF16),
            pltpu.VMEM((N_DEV, RB, DM), BF16),
            pltpu.SemaphoreType.DMA((2,)),
            pltpu.SemaphoreType.DMA((N_DEV,)),
            pltpu.SemaphoreType.DMA((N_DEV,)),
            pltpu.SemaphoreType.DMA((2,)),
            pltpu.SemaphoreType.DMA((2,)),
            pltpu.SemaphoreType.DMA((4,)),
            pltpu.SemaphoreType.DMA((2,)),
            pltpu.SemaphoreType.DMA((N_DEV,)),
            pltpu.SemaphoreType.DMA((N_DEV,)),
            pltpu.SemaphoreType.DMA((N_DEV,)),
            pltpu.SemaphoreType.DMA((N_DEV,)),
        ],
        compiler_params=pltpu.CompilerParams(collective_id=0),
    )(x, Wq, K_ext, V_ext, Wo)
